# baseline (speedup 1.0000x reference)
import functools

import jax
import jax.numpy as jnp
from jax import lax
from jax.experimental import pallas as pl
from jax.experimental.pallas import tpu as pltpu

H = 4
B = 16
TOPK = 10
BN = 64


def _sc_gather_pair(rna_table, rna_idx, atac_table, atac_idx, n_rows, width):
    from jax.experimental.pallas import tpu_sc as plsc

    info = plsc.get_sparse_core_info()
    nw = info.num_cores * info.num_subcores
    b_per_w = n_rows // nw
    mesh = plsc.VectorSubcoreMesh(core_axis_name="c", subcore_axis_name="s")

    @functools.partial(
        pl.kernel,
        mesh=mesh,
        compiler_params=pltpu.CompilerParams(use_tc_tiling_on_sc=False),
        out_type=[
            jax.ShapeDtypeStruct((n_rows, width), jnp.float32),
            jax.ShapeDtypeStruct((n_rows, width), jnp.float32),
        ],
        scratch_types=[
            pltpu.VMEM((b_per_w,), jnp.int32),
            pltpu.VMEM((b_per_w,), jnp.int32),
            pltpu.VMEM((b_per_w, width), jnp.float32),
            pltpu.VMEM((b_per_w, width), jnp.float32),
            pltpu.SemaphoreType.DMA,
            pltpu.SemaphoreType.DMA,
        ],
    )
    def gather_k(rt, ri, at, ai, out_r, out_a,
                 idx_r, idx_a, rows_r, rows_a, sem_r, sem_a):
        wid = lax.axis_index("s") * info.num_cores + lax.axis_index("c")
        base = wid * b_per_w
        pltpu.sync_copy(ri.at[pl.ds(base, b_per_w)], idx_r)
        pltpu.sync_copy(ai.at[pl.ds(base, b_per_w)], idx_a)
        cp_r = pltpu.async_copy(rt.at[idx_r], rows_r, sem_r)
        cp_a = pltpu.async_copy(at.at[idx_a], rows_a, sem_a)
        cp_r.wait()
        pltpu.sync_copy(rows_r, out_r.at[pl.ds(base, b_per_w)])
        cp_a.wait()
        pltpu.sync_copy(rows_a, out_a.at[pl.ds(base, b_per_w)])

    return gather_k(rna_table, rna_idx, atac_table, atac_idx)


def _proj_body(emb_ref, feat_ref, wq_ref, wv_ref, ws_ref,
               bq_ref, bv_ref, bs_ref, o1_ref, o2_ref):
    hid = ws_ref.shape[1]
    ed = feat_ref.shape[1]
    x = jnp.concatenate([emb_ref[:, :ed], feat_ref[...]], axis=1)
    o1_ref[:, :hid] = (
        jnp.dot(x, wq_ref[...], preferred_element_type=jnp.float32)
        + bq_ref[...])
    o1_ref[:, hid:] = (
        jnp.dot(x, wv_ref[...], preferred_element_type=jnp.float32)
        + bv_ref[...])
    o2_ref[...] = (
        jnp.dot(x, ws_ref[...], preferred_element_type=jnp.float32)
        + bs_ref[...])


def _project(emb, feat, wq, wv, ws, bq, bv, bs):
    n, hid = feat.shape[0], wq.shape[1]
    return pl.pallas_call(
        _proj_body,
        out_shape=[jax.ShapeDtypeStruct((n, 2 * hid), jnp.float32),
                   jax.ShapeDtypeStruct((n, hid), jnp.float32)],
    )(emb, feat, wq, wv, ws,
      bq.reshape(1, hid), bv.reshape(1, hid), bs.reshape(1, hid))


def _attn_body(q_ref, k_ref, va_ref, vr_ref, m_ref, r2a_ref, a2r_ref):
    i = pl.program_id(1)
    na = k_ref.shape[0]
    q = q_ref[...]
    k = k_ref[...]
    logits = lax.dot_general(
        q, k, (((1,), (1,)), ((), ())), preferred_element_type=jnp.float32
    )
    logits = logits * m_ref[0]

    @pl.when(i == 0)
    def _():
        a2r_ref[0] = jnp.zeros_like(a2r_ref[0])

    blockmax = jnp.max(logits)
    anyhit = (1.0 / (1.0 + jnp.exp(-blockmax))) > 0.8

    @pl.when(jnp.logical_not(anyhit))
    def _():
        r2a_ref[0] = jnp.zeros_like(r2a_ref[0])

    @pl.when(anyhit)
    def _():
        col = lax.broadcasted_iota(jnp.int32, (BN, na), 1)
        work = logits
        sig0 = None
        denom = None
        u = jnp.zeros((BN, na), jnp.float32)
        for j in range(TOPK):
            m = jnp.max(work, axis=1, keepdims=True)
            eq = work == m
            idx = jnp.min(jnp.where(eq, col, na), axis=1, keepdims=True)
            sel = col == idx
            work = jnp.where(sel, -jnp.inf, work)
            sig = 1.0 / (1.0 + jnp.exp(-m))
            if j == 0:
                sig0 = sig
                e = jnp.ones_like(sig)
                denom = e
            else:
                e = jnp.exp(sig - sig0)
                denom = denom + e
            u = jnp.where(sel, e * (sig > 0.8).astype(jnp.float32), u)

        inv_d = 1.0 / denom
        r2a_ref[0] = lax.dot_general(
            u, va_ref[...], (((1,), (0,)), ((), ())),
            preferred_element_type=jnp.float32,
        ) * inv_d
        a2r_ref[0] = a2r_ref[0] + lax.dot_general(
            u, vr_ref[...] * inv_d, (((0,), (0,)), ((), ())),
            preferred_element_type=jnp.float32,
        )


def _attention(proj_r, proj_a, chrom_mask, nr, na, dh):
    nb = nr // BN
    grid = (H, nb)
    return pl.pallas_call(
        _attn_body,
        grid=grid,
        in_specs=[
            pl.BlockSpec((BN, dh), lambda h, i: (i, h)),
            pl.BlockSpec((na, dh), lambda h, i: (0, h)),
            pl.BlockSpec((na, dh), lambda h, i: (0, H + h)),
            pl.BlockSpec((BN, dh), lambda h, i: (i, H + h)),
            pl.BlockSpec((1, BN, na), lambda h, i: (h, i, 0)),
        ],
        out_specs=[
            pl.BlockSpec((1, BN, dh), lambda h, i: (h, i, 0)),
            pl.BlockSpec((1, na, dh), lambda h, i: (h, 0, 0)),
        ],
        out_shape=[
            jax.ShapeDtypeStruct((H, nr, dh), jnp.float32),
            jax.ShapeDtypeStruct((H, na, dh), jnp.float32),
        ],
    )(proj_r, proj_a, proj_a, proj_r, chrom_mask)


def _tail_body(r2a_ref, a2r_ref, pr_ref, pa_ref, segr_ref, sega_ref,
               wor_ref, bor_ref, woa_ref, boa_ref,
               wdr_ref, bdr_ref, wda_ref, bda_ref,
               w1_ref, b1_ref, w2_ref, b2_ref, w3_ref, b3_ref,
               o_ref):
    dh = r2a_ref.shape[2]
    hid = wor_ref.shape[1]

    def headsum(src_ref, w_ref):
        acc = None
        for h in range(H):
            part = lax.dot_general(
                src_ref[h], w_ref[h * dh:(h + 1) * dh, :],
                (((1,), (0,)), ((), ())), preferred_element_type=jnp.float32)
            acc = part if acc is None else acc + part
        return acc

    hr = headsum(r2a_ref, wor_ref) + bor_ref[...]
    ha = headsum(a2r_ref, woa_ref) + boa_ref[...]
    sr = pr_ref[...]
    sa = pa_ref[...]
    out_r = (jnp.dot(hr, wdr_ref[:hid], preferred_element_type=jnp.float32)
             + jnp.dot(sr, wdr_ref[hid:], preferred_element_type=jnp.float32)
             + bdr_ref[...])
    out_a = (jnp.dot(ha, wda_ref[:hid], preferred_element_type=jnp.float32)
             + jnp.dot(sa, wda_ref[hid:], preferred_element_type=jnp.float32)
             + bda_ref[...])

    def pool(x, seg_ref):
        n = x.shape[0]
        bi = lax.broadcasted_iota(jnp.int32, (B, n), 0)
        ind = (bi == seg_ref[...]).astype(jnp.float32)
        s = jnp.dot(ind, x, preferred_element_type=jnp.float32)
        c = jnp.sum(ind, axis=1, keepdims=True)
        return s / jnp.maximum(c, 1.0)

    pr = pool(out_r, segr_ref)
    pa = pool(out_a, sega_ref)
    h1 = jnp.maximum(
        jnp.dot(pr, w1_ref[:hid], preferred_element_type=jnp.float32)
        + jnp.dot(pa, w1_ref[hid:], preferred_element_type=jnp.float32)
        + b1_ref[...], 0.0)
    h2 = jnp.maximum(
        jnp.dot(h1, w2_ref[...], preferred_element_type=jnp.float32)
        + b2_ref[...], 0.0)
    o_ref[...] = (jnp.dot(h2, w3_ref[...], preferred_element_type=jnp.float32)
                  + b3_ref[...])


def _tail(r2a, a2r, proj_r, proj_a, rna_batch, atac_batch,
          W_or, b_or, W_oa, b_oa,
          W_dr, b_dr, W_da, b_da, W1, b1, W2, b2, W3, b3):
    hid = W_or.shape[1]
    w3p = jnp.pad(W3, ((0, 0), (0, 127)))
    b3p = jnp.pad(b3, (0, 127))
    y = pl.pallas_call(
        _tail_body,
        out_shape=jax.ShapeDtypeStruct((B, 128), jnp.float32),
    )(r2a, a2r, proj_r, proj_a,
      rna_batch.reshape(1, -1).astype(jnp.int32),
      atac_batch.reshape(1, -1).astype(jnp.int32),
      W_or, b_or.reshape(1, -1), W_oa, b_oa.reshape(1, -1),
      W_dr, b_dr.reshape(1, -1), W_da, b_da.reshape(1, -1),
      W1, b1.reshape(1, -1),
      W2, b2.reshape(1, -1), w3p, b3p.reshape(1, -1))
    return y[:, :1]


def kernel(x_rna_feat, x_atac_feat, chrom_mask, rna_id, atac_id, rna_batch,
           atac_batch, rna_table, atac_table, W_q, b_q, W_k, b_k, W_vr, b_vr,
           W_va, b_va, W_or, b_or, W_oa, b_oa, W_sr, b_sr, W_sa, b_sa,
           W_dr, b_dr, W_da, b_da, W1, b1, W2, b2, W3, b3):
    nr, na = x_rna_feat.shape[0], x_atac_feat.shape[0]
    ed = rna_table.shape[1]
    hid = W_q.shape[1]
    dh = hid // H

    rid = rna_id.astype(jnp.int32)
    aid = atac_id.astype(jnp.int32)
    g_r, g_a = _sc_gather_pair(rna_table, rid, atac_table, aid, nr, ed)

    proj_r, skip_r = _project(g_r, x_rna_feat, W_q, W_vr, W_sr,
                              b_q, b_vr, b_sr)
    proj_a, skip_a = _project(g_a, x_atac_feat, W_k, W_va, W_sa,
                              b_k, b_va, b_sa)

    r2a, a2r = _attention(proj_r, proj_a, chrom_mask, nr, na, dh)

    return _tail(r2a, a2r, skip_r, skip_a,
                 rna_batch, atac_batch, W_or, b_or, W_oa, b_oa,
                 W_dr, b_dr, W_da, b_da, W1, b1, W2, b2, W3, b3)


# speedup vs baseline: 1.2371x; 1.2371x over previous
import functools

import jax
import jax.numpy as jnp
from jax import lax
from jax.experimental import pallas as pl
from jax.experimental.pallas import tpu as pltpu

H = 4
B = 16
TOPK = 10
BN = 128


def _sc_gather_pair(rna_table, rna_idx, atac_table, atac_idx, n_rows, width):
    from jax.experimental.pallas import tpu_sc as plsc

    info = plsc.get_sparse_core_info()
    nw = info.num_cores * info.num_subcores
    b_per_w = n_rows // nw
    mesh = plsc.VectorSubcoreMesh(core_axis_name="c", subcore_axis_name="s")

    @functools.partial(
        pl.kernel,
        mesh=mesh,
        compiler_params=pltpu.CompilerParams(use_tc_tiling_on_sc=False),
        out_type=[
            jax.ShapeDtypeStruct((n_rows, width), jnp.float32),
            jax.ShapeDtypeStruct((n_rows, width), jnp.float32),
        ],
        scratch_types=[
            pltpu.VMEM((b_per_w,), jnp.int32),
            pltpu.VMEM((b_per_w,), jnp.int32),
            pltpu.VMEM((b_per_w, width), jnp.float32),
            pltpu.VMEM((b_per_w, width), jnp.float32),
            pltpu.SemaphoreType.DMA,
            pltpu.SemaphoreType.DMA,
        ],
    )
    def gather_k(rt, ri, at, ai, out_r, out_a,
                 idx_r, idx_a, rows_r, rows_a, sem_r, sem_a):
        wid = lax.axis_index("s") * info.num_cores + lax.axis_index("c")
        base = wid * b_per_w
        pltpu.sync_copy(ri.at[pl.ds(base, b_per_w)], idx_r)
        pltpu.sync_copy(ai.at[pl.ds(base, b_per_w)], idx_a)
        cp_r = pltpu.async_copy(rt.at[idx_r], rows_r, sem_r)
        cp_a = pltpu.async_copy(at.at[idx_a], rows_a, sem_a)
        cp_r.wait()
        pltpu.sync_copy(rows_r, out_r.at[pl.ds(base, b_per_w)])
        cp_a.wait()
        pltpu.sync_copy(rows_a, out_a.at[pl.ds(base, b_per_w)])

    return gather_k(rna_table, rna_idx, atac_table, atac_idx)


def _fused_body(embr_ref, featr_ref, emba_ref, feata_ref,
                wq_ref, wvr_ref, wsr_ref, bq_ref, bvr_ref, bsr_ref,
                wk_ref, wva_ref, wsa_ref, bk_ref, bva_ref, bsa_ref,
                m_ref, segr_ref, sega_ref,
                wor_ref, bor_ref, woa_ref, boa_ref,
                wdr_ref, bdr_ref, wda_ref, bda_ref,
                w1_ref, b1_ref, w2_ref, b2_ref, w3_ref, b3_ref,
                o_ref,
                pr_s, pa_s, r2a_s, a2r_s):
    h = pl.program_id(0)
    i = pl.program_id(1)
    nb = pl.num_programs(1)
    na = pa_s.shape[1]
    dh = pr_s.shape[2]

    @pl.when(jnp.logical_and(h == 0, i == 0))
    def _():
        x_r = jnp.concatenate([embr_ref[...], featr_ref[...]], axis=1)
        x_a = jnp.concatenate([emba_ref[...], feata_ref[...]], axis=1)
        for b in range(H):
            sl = slice(b * dh, (b + 1) * dh)
            pr_s[b] = jnp.dot(x_r, wq_ref[:, sl],
                              preferred_element_type=jnp.float32) + bq_ref[:, sl]
            pr_s[H + b] = jnp.dot(x_r, wvr_ref[:, sl],
                                  preferred_element_type=jnp.float32) + bvr_ref[:, sl]
            pa_s[b] = jnp.dot(x_a, wk_ref[:, sl],
                              preferred_element_type=jnp.float32) + bk_ref[:, sl]
            pa_s[H + b] = jnp.dot(x_a, wva_ref[:, sl],
                                  preferred_element_type=jnp.float32) + bva_ref[:, sl]
        a2r_s[...] = jnp.zeros_like(a2r_s[...])

    q = pr_s[h, pl.ds(i * BN, BN), :]
    k = pa_s[h]
    logits = lax.dot_general(
        q, k, (((1,), (1,)), ((), ())), preferred_element_type=jnp.float32
    )
    logits = logits * m_ref[0]

    blockmax = jnp.max(logits)
    anyhit = (1.0 / (1.0 + jnp.exp(-blockmax))) > 0.8

    @pl.when(jnp.logical_not(anyhit))
    def _():
        r2a_s[h, pl.ds(i * BN, BN), :] = jnp.zeros((BN, dh), jnp.float32)

    @pl.when(anyhit)
    def _():
        col = lax.broadcasted_iota(jnp.int32, (BN, na), 1)
        work = logits
        sig0 = None
        denom = None
        u = jnp.zeros((BN, na), jnp.float32)
        for j in range(TOPK):
            m = jnp.max(work, axis=1, keepdims=True)
            eq = work == m
            idx = jnp.min(jnp.where(eq, col, na), axis=1, keepdims=True)
            sel = col == idx
            work = jnp.where(sel, -jnp.inf, work)
            sig = 1.0 / (1.0 + jnp.exp(-m))
            if j == 0:
                sig0 = sig
                e = jnp.ones_like(sig)
                denom = e
            else:
                e = jnp.exp(sig - sig0)
                denom = denom + e
            u = jnp.where(sel, e * (sig > 0.8).astype(jnp.float32), u)

        inv_d = 1.0 / denom
        va = pa_s[H + h]
        vr = pr_s[H + h, pl.ds(i * BN, BN), :]
        r2a_s[h, pl.ds(i * BN, BN), :] = lax.dot_general(
            u, va, (((1,), (0,)), ((), ())),
            preferred_element_type=jnp.float32,
        ) * inv_d
        a2r_s[h] = a2r_s[h] + lax.dot_general(
            u, vr * inv_d, (((0,), (0,)), ((), ())),
            preferred_element_type=jnp.float32,
        )

    @pl.when(jnp.logical_and(h == H - 1, i == nb - 1))
    def _():
        hid = wor_ref.shape[1]

        def headsum(src_s, w_ref):
            acc = None
            for hh in range(H):
                part = lax.dot_general(
                    src_s[hh], w_ref[hh * dh:(hh + 1) * dh, :],
                    (((1,), (0,)), ((), ())),
                    preferred_element_type=jnp.float32)
                acc = part if acc is None else acc + part
            return acc

        hr = headsum(r2a_s, wor_ref) + bor_ref[...]
        ha = headsum(a2r_s, woa_ref) + boa_ref[...]
        x_r = jnp.concatenate([embr_ref[...], featr_ref[...]], axis=1)
        x_a = jnp.concatenate([emba_ref[...], feata_ref[...]], axis=1)
        sr = jnp.dot(x_r, wsr_ref[...],
                     preferred_element_type=jnp.float32) + bsr_ref[...]
        sa = jnp.dot(x_a, wsa_ref[...],
                     preferred_element_type=jnp.float32) + bsa_ref[...]
        out_r = (jnp.dot(hr, wdr_ref[:hid], preferred_element_type=jnp.float32)
                 + jnp.dot(sr, wdr_ref[hid:],
                           preferred_element_type=jnp.float32)
                 + bdr_ref[...])
        out_a = (jnp.dot(ha, wda_ref[:hid], preferred_element_type=jnp.float32)
                 + jnp.dot(sa, wda_ref[hid:],
                           preferred_element_type=jnp.float32)
                 + bda_ref[...])

        def pool(x, seg_ref):
            n = x.shape[0]
            bi = lax.broadcasted_iota(jnp.int32, (B, n), 0)
            ind = (bi == seg_ref[...]).astype(jnp.float32)
            s = jnp.dot(ind, x, preferred_element_type=jnp.float32)
            c = jnp.sum(ind, axis=1, keepdims=True)
            return s / jnp.maximum(c, 1.0)

        pr = pool(out_r, segr_ref)
        pa = pool(out_a, sega_ref)
        h1 = jnp.maximum(
            jnp.dot(pr, w1_ref[:hid], preferred_element_type=jnp.float32)
            + jnp.dot(pa, w1_ref[hid:], preferred_element_type=jnp.float32)
            + b1_ref[...], 0.0)
        h2 = jnp.maximum(
            jnp.dot(h1, w2_ref[...], preferred_element_type=jnp.float32)
            + b2_ref[...], 0.0)
        o_ref[...] = (jnp.dot(h2, w3_ref[...],
                              preferred_element_type=jnp.float32)
                      + b3_ref[...])


def _fused(emb_r, x_rna_feat, emb_a, x_atac_feat, chrom_mask,
           rna_batch, atac_batch,
           W_q, b_q, W_k, b_k, W_vr, b_vr, W_va, b_va, W_or, b_or,
           W_oa, b_oa, W_sr, b_sr, W_sa, b_sa, W_dr, b_dr, W_da, b_da,
           W1, b1, W2, b2, W3, b3):
    nr, na = x_rna_feat.shape[0], x_atac_feat.shape[0]
    hid = W_q.shape[1]
    dh = hid // H
    nb = nr // BN
    w3p = jnp.pad(W3, ((0, 0), (0, 127)))
    b3p = jnp.pad(b3, (0, 127))

    def full2(a):
        return pl.BlockSpec(a.shape, lambda h, i: tuple(0 for _ in a.shape))

    r1 = lambda b: b.reshape(1, -1)
    ins = [emb_r, x_rna_feat, emb_a, x_atac_feat,
           W_q, W_vr, W_sr, r1(b_q), r1(b_vr), r1(b_sr),
           W_k, W_va, W_sa, r1(b_k), r1(b_va), r1(b_sa)]
    in_specs = [full2(a) for a in ins]
    ins.append(chrom_mask)
    in_specs.append(pl.BlockSpec((1, BN, na), lambda h, i: (h, i, 0)))
    rest = [rna_batch.reshape(1, -1).astype(jnp.int32),
            atac_batch.reshape(1, -1).astype(jnp.int32),
            W_or, r1(b_or), W_oa, r1(b_oa),
            W_dr, r1(b_dr), W_da, r1(b_da),
            W1, r1(b1), W2, r1(b2), w3p, r1(b3p)]
    ins.extend(rest)
    in_specs.extend(full2(a) for a in rest)

    y = pl.pallas_call(
        _fused_body,
        grid=(H, nb),
        in_specs=in_specs,
        out_specs=pl.BlockSpec((B, 128), lambda h, i: (0, 0)),
        out_shape=jax.ShapeDtypeStruct((B, 128), jnp.float32),
        scratch_shapes=[
            pltpu.VMEM((2 * H, nr, dh), jnp.float32),
            pltpu.VMEM((2 * H, na, dh), jnp.float32),
            pltpu.VMEM((H, nr, dh), jnp.float32),
            pltpu.VMEM((H, na, dh), jnp.float32),
        ],
    )(*ins)
    return y[:, :1]


def kernel(x_rna_feat, x_atac_feat, chrom_mask, rna_id, atac_id, rna_batch,
           atac_batch, rna_table, atac_table, W_q, b_q, W_k, b_k, W_vr, b_vr,
           W_va, b_va, W_or, b_or, W_oa, b_oa, W_sr, b_sr, W_sa, b_sa,
           W_dr, b_dr, W_da, b_da, W1, b1, W2, b2, W3, b3):
    nr = x_rna_feat.shape[0]
    ed = rna_table.shape[1]

    rid = rna_id.astype(jnp.int32)
    aid = atac_id.astype(jnp.int32)
    g_r, g_a = _sc_gather_pair(rna_table, rid, atac_table, aid, nr, ed)

    return _fused(g_r, x_rna_feat, g_a, x_atac_feat, chrom_mask,
                  rna_batch, atac_batch,
                  W_q, b_q, W_k, b_k, W_vr, b_vr, W_va, b_va, W_or, b_or,
                  W_oa, b_oa, W_sr, b_sr, W_sa, b_sa, W_dr, b_dr, W_da, b_da,
                  W1, b1, W2, b2, W3, b3)


# speedup vs baseline: 1.2997x; 1.0506x over previous
import functools

import jax
import jax.numpy as jnp
from jax import lax
from jax.experimental import pallas as pl
from jax.experimental.pallas import tpu as pltpu

H = 4
B = 16
TOPK = 10
BN = 128


def _sc_gather_pair(rna_table, rna_idx, atac_table, atac_idx, n_rows, width):
    from jax.experimental.pallas import tpu_sc as plsc

    info = plsc.get_sparse_core_info()
    nw = info.num_cores * info.num_subcores
    b_per_w = n_rows // nw
    mesh = plsc.VectorSubcoreMesh(core_axis_name="c", subcore_axis_name="s")

    @functools.partial(
        pl.kernel,
        mesh=mesh,
        compiler_params=pltpu.CompilerParams(use_tc_tiling_on_sc=False),
        out_type=[
            jax.ShapeDtypeStruct((n_rows, width), jnp.float32),
            jax.ShapeDtypeStruct((n_rows, width), jnp.float32),
        ],
        scratch_types=[
            pltpu.VMEM((b_per_w,), jnp.int32),
            pltpu.VMEM((b_per_w,), jnp.int32),
            pltpu.VMEM((b_per_w, width), jnp.float32),
            pltpu.VMEM((b_per_w, width), jnp.float32),
            pltpu.SemaphoreType.DMA,
            pltpu.SemaphoreType.DMA,
        ],
    )
    def gather_k(rt, ri, at, ai, out_r, out_a,
                 idx_r, idx_a, rows_r, rows_a, sem_r, sem_a):
        wid = lax.axis_index("s") * info.num_cores + lax.axis_index("c")
        base = wid * b_per_w
        pltpu.sync_copy(ri.at[pl.ds(base, b_per_w)], idx_r)
        pltpu.sync_copy(ai.at[pl.ds(base, b_per_w)], idx_a)
        cp_r = pltpu.async_copy(rt.at[idx_r], rows_r, sem_r)
        cp_a = pltpu.async_copy(at.at[idx_a], rows_a, sem_a)
        cp_r.wait()
        pltpu.sync_copy(rows_r, out_r.at[pl.ds(base, b_per_w)])
        cp_a.wait()
        pltpu.sync_copy(rows_a, out_a.at[pl.ds(base, b_per_w)])

    return gather_k(rna_table, rna_idx, atac_table, atac_idx)


def _fused_body(embr_ref, featr_ref, emba_ref, feata_ref,
                wq_ref, wvr_ref, wsr_ref, bq_ref, bvr_ref, bsr_ref,
                wk_ref, wva_ref, wsa_ref, bk_ref, bva_ref, bsa_ref,
                m_ref, segr_ref, sega_ref,
                wor_ref, bor_ref, woa_ref, boa_ref,
                wdr_ref, bdr_ref, wda_ref, bda_ref,
                w1_ref, b1_ref, w2_ref, b2_ref, w3_ref, b3_ref,
                o_ref,
                pr_s, pa_s, r2a_s, a2r_s):
    h = pl.program_id(0)
    i = pl.program_id(1)
    nb = pl.num_programs(1)
    na = pa_s.shape[1]
    dh = pr_s.shape[2]

    @pl.when(jnp.logical_and(h == 0, i == 0))
    def _():
        x_r = jnp.concatenate([embr_ref[...], featr_ref[...]], axis=1)
        x_a = jnp.concatenate([emba_ref[...], feata_ref[...]], axis=1)
        for b in range(H):
            sl = slice(b * dh, (b + 1) * dh)
            pr_s[b] = jnp.dot(x_r, wq_ref[:, sl],
                              preferred_element_type=jnp.float32) + bq_ref[:, sl]
            pr_s[H + b] = jnp.dot(x_r, wvr_ref[:, sl],
                                  preferred_element_type=jnp.float32) + bvr_ref[:, sl]
            pa_s[b] = jnp.dot(x_a, wk_ref[:, sl],
                              preferred_element_type=jnp.float32) + bk_ref[:, sl]
            pa_s[H + b] = jnp.dot(x_a, wva_ref[:, sl],
                                  preferred_element_type=jnp.float32) + bva_ref[:, sl]
        a2r_s[...] = jnp.zeros_like(a2r_s[...])

    q = pr_s[h, pl.ds(i * BN, BN), :]
    k = pa_s[h]
    logits = lax.dot_general(
        q, k, (((1,), (1,)), ((), ())), preferred_element_type=jnp.float32
    )
    logits = logits * m_ref[0]

    blockmax = jnp.max(logits)
    anyhit = (1.0 / (1.0 + jnp.exp(-blockmax))) > 0.8

    @pl.when(jnp.logical_not(anyhit))
    def _():
        r2a_s[h, pl.ds(i * BN, BN), :] = jnp.zeros((BN, dh), jnp.float32)

    @pl.when(anyhit)
    def _():
        SB = BN // 2
        for g in range(2):
            sub = logits[g * SB:(g + 1) * SB, :]
            submax = jnp.max(sub)
            subhit = (1.0 / (1.0 + jnp.exp(-submax))) > 0.8
            base = i * BN + g * SB

            @pl.when(jnp.logical_not(subhit))
            def _():
                r2a_s[h, pl.ds(base, SB), :] = jnp.zeros((SB, dh),
                                                         jnp.float32)

            @pl.when(subhit)
            def _():
                col = lax.broadcasted_iota(jnp.int32, (SB, na), 1)
                work = sub
                sig0 = None
                denom = None
                u = jnp.zeros((SB, na), jnp.float32)
                for j in range(TOPK):
                    m = jnp.max(work, axis=1, keepdims=True)
                    eq = work == m
                    idx = jnp.min(jnp.where(eq, col, na), axis=1,
                                  keepdims=True)
                    sel = col == idx
                    work = jnp.where(sel, -jnp.inf, work)
                    sig = 1.0 / (1.0 + jnp.exp(-m))
                    if j == 0:
                        sig0 = sig
                        e = jnp.ones_like(sig)
                        denom = e
                    else:
                        e = jnp.exp(sig - sig0)
                        denom = denom + e
                    u = jnp.where(sel,
                                  e * (sig > 0.8).astype(jnp.float32), u)

                inv_d = 1.0 / denom
                va = pa_s[H + h]
                vr = pr_s[H + h, pl.ds(base, SB), :]
                r2a_s[h, pl.ds(base, SB), :] = lax.dot_general(
                    u, va, (((1,), (0,)), ((), ())),
                    preferred_element_type=jnp.float32,
                ) * inv_d
                a2r_s[h] = a2r_s[h] + lax.dot_general(
                    u, vr * inv_d, (((0,), (0,)), ((), ())),
                    preferred_element_type=jnp.float32,
                )

    @pl.when(jnp.logical_and(h == H - 1, i == nb - 1))
    def _():
        hid = wor_ref.shape[1]

        def headsum(src_s, w_ref):
            acc = None
            for hh in range(H):
                part = lax.dot_general(
                    src_s[hh], w_ref[hh * dh:(hh + 1) * dh, :],
                    (((1,), (0,)), ((), ())),
                    preferred_element_type=jnp.float32)
                acc = part if acc is None else acc + part
            return acc

        hr = headsum(r2a_s, wor_ref) + bor_ref[...]
        ha = headsum(a2r_s, woa_ref) + boa_ref[...]
        x_r = jnp.concatenate([embr_ref[...], featr_ref[...]], axis=1)
        x_a = jnp.concatenate([emba_ref[...], feata_ref[...]], axis=1)
        sr = jnp.dot(x_r, wsr_ref[...],
                     preferred_element_type=jnp.float32) + bsr_ref[...]
        sa = jnp.dot(x_a, wsa_ref[...],
                     preferred_element_type=jnp.float32) + bsa_ref[...]
        out_r = (jnp.dot(hr, wdr_ref[:hid], preferred_element_type=jnp.float32)
                 + jnp.dot(sr, wdr_ref[hid:],
                           preferred_element_type=jnp.float32)
                 + bdr_ref[...])
        out_a = (jnp.dot(ha, wda_ref[:hid], preferred_element_type=jnp.float32)
                 + jnp.dot(sa, wda_ref[hid:],
                           preferred_element_type=jnp.float32)
                 + bda_ref[...])

        def pool(x, seg_ref):
            n = x.shape[0]
            bi = lax.broadcasted_iota(jnp.int32, (B, n), 0)
            ind = (bi == seg_ref[...]).astype(jnp.float32)
            s = jnp.dot(ind, x, preferred_element_type=jnp.float32)
            c = jnp.sum(ind, axis=1, keepdims=True)
            return s / jnp.maximum(c, 1.0)

        pr = pool(out_r, segr_ref)
        pa = pool(out_a, sega_ref)
        h1 = jnp.maximum(
            jnp.dot(pr, w1_ref[:hid], preferred_element_type=jnp.float32)
            + jnp.dot(pa, w1_ref[hid:], preferred_element_type=jnp.float32)
            + b1_ref[...], 0.0)
        h2 = jnp.maximum(
            jnp.dot(h1, w2_ref[...], preferred_element_type=jnp.float32)
            + b2_ref[...], 0.0)
        o_ref[...] = (jnp.dot(h2, w3_ref[...],
                              preferred_element_type=jnp.float32)
                      + b3_ref[...])


def _fused(emb_r, x_rna_feat, emb_a, x_atac_feat, chrom_mask,
           rna_batch, atac_batch,
           W_q, b_q, W_k, b_k, W_vr, b_vr, W_va, b_va, W_or, b_or,
           W_oa, b_oa, W_sr, b_sr, W_sa, b_sa, W_dr, b_dr, W_da, b_da,
           W1, b1, W2, b2, W3, b3):
    nr, na = x_rna_feat.shape[0], x_atac_feat.shape[0]
    hid = W_q.shape[1]
    dh = hid // H
    nb = nr // BN
    w3p = jnp.pad(W3, ((0, 0), (0, 127)))
    b3p = jnp.pad(b3, (0, 127))

    def full2(a):
        return pl.BlockSpec(a.shape, lambda h, i: tuple(0 for _ in a.shape))

    r1 = lambda b: b.reshape(1, -1)
    ins = [emb_r, x_rna_feat, emb_a, x_atac_feat,
           W_q, W_vr, W_sr, r1(b_q), r1(b_vr), r1(b_sr),
           W_k, W_va, W_sa, r1(b_k), r1(b_va), r1(b_sa)]
    in_specs = [full2(a) for a in ins]
    ins.append(chrom_mask)
    in_specs.append(pl.BlockSpec((1, BN, na), lambda h, i: (h, i, 0)))
    rest = [rna_batch.reshape(1, -1).astype(jnp.int32),
            atac_batch.reshape(1, -1).astype(jnp.int32),
            W_or, r1(b_or), W_oa, r1(b_oa),
            W_dr, r1(b_dr), W_da, r1(b_da),
            W1, r1(b1), W2, r1(b2), w3p, r1(b3p)]
    ins.extend(rest)
    in_specs.extend(full2(a) for a in rest)

    y = pl.pallas_call(
        _fused_body,
        grid=(H, nb),
        in_specs=in_specs,
        out_specs=pl.BlockSpec((B, 128), lambda h, i: (0, 0)),
        out_shape=jax.ShapeDtypeStruct((B, 128), jnp.float32),
        scratch_shapes=[
            pltpu.VMEM((2 * H, nr, dh), jnp.float32),
            pltpu.VMEM((2 * H, na, dh), jnp.float32),
            pltpu.VMEM((H, nr, dh), jnp.float32),
            pltpu.VMEM((H, na, dh), jnp.float32),
        ],
    )(*ins)
    return y[:, :1]


def kernel(x_rna_feat, x_atac_feat, chrom_mask, rna_id, atac_id, rna_batch,
           atac_batch, rna_table, atac_table, W_q, b_q, W_k, b_k, W_vr, b_vr,
           W_va, b_va, W_or, b_or, W_oa, b_oa, W_sr, b_sr, W_sa, b_sa,
           W_dr, b_dr, W_da, b_da, W1, b1, W2, b2, W3, b3):
    nr = x_rna_feat.shape[0]
    ed = rna_table.shape[1]

    rid = rna_id.astype(jnp.int32)
    aid = atac_id.astype(jnp.int32)
    g_r, g_a = _sc_gather_pair(rna_table, rid, atac_table, aid, nr, ed)

    return _fused(g_r, x_rna_feat, g_a, x_atac_feat, chrom_mask,
                  rna_batch, atac_batch,
                  W_q, b_q, W_k, b_k, W_vr, b_vr, W_va, b_va, W_or, b_or,
                  W_oa, b_oa, W_sr, b_sr, W_sa, b_sa, W_dr, b_dr, W_da, b_da,
                  W1, b1, W2, b2, W3, b3)
